# Initial kernel scaffold; baseline (speedup 1.0000x reference)
import jax
import jax.numpy as jnp
from jax.experimental import pallas as pl


def kernel(x, table):
    raise NotImplementedError("write your pallas kernel here")


# speedup vs baseline: 1.4789x; 1.4789x over previous
import functools

import jax
import jax.numpy as jnp
from jax import lax
from jax.experimental import pallas as pl
from jax.experimental.pallas import tpu as pltpu
from jax.experimental.pallas import tpu_sc as plsc


@functools.cache
def _build_gather(N, V, D, NC, NS):
    NW = NC * NS
    per_w = N // NW
    CHUNK = 1600
    n_chunks = per_w // CHUNK
    assert per_w % CHUNK == 0 and CHUNK % 8 == 0

    mesh = plsc.VectorSubcoreMesh(core_axis_name="c", subcore_axis_name="s")

    @functools.partial(
        pl.kernel,
        out_type=jax.ShapeDtypeStruct((N, D), jnp.float32),
        mesh=mesh,
        scratch_types=[
            pltpu.VMEM((CHUNK,), jnp.int32),
            pltpu.VMEM((CHUNK, D), jnp.float32),
            pltpu.SemaphoreType.DMA,
        ],
        compiler_params=pltpu.CompilerParams(use_tc_tiling_on_sc=False),
    )
    def gather_kernel(x_hbm, table_hbm, out_hbm, idx_v, rows_v, sem):
        wid = lax.axis_index("s") * NC + lax.axis_index("c")
        base = wid * per_w

        def body(i, carry):
            off = base + i * CHUNK
            pltpu.sync_copy(x_hbm.at[pl.ds(off, CHUNK)], idx_v)
            pltpu.async_copy(table_hbm.at[idx_v], rows_v, sem).wait()
            pltpu.sync_copy(rows_v, out_hbm.at[pl.ds(off, CHUNK)])
            return carry

        lax.fori_loop(0, n_chunks, body, 0)

    return gather_kernel


def kernel(x, table):
    B, S = x.shape
    V, D = table.shape
    N = B * S
    info = plsc.get_sparse_core_info()
    f = _build_gather(N, V, D, info.num_cores, info.num_subcores)
    out = f(x.reshape(N), table)
    return out.reshape(B, S, D)


# speedup vs baseline: 1.4923x; 1.0091x over previous
import functools

import jax
import jax.numpy as jnp
from jax import lax
from jax.experimental import pallas as pl
from jax.experimental.pallas import tpu as pltpu
from jax.experimental.pallas import tpu_sc as plsc

NBUF = 2


@functools.cache
def _build_gather(N, V, D, NC, NS):
    NW = NC * NS
    per_w = N // NW
    CHUNK = 1600
    n_chunks = per_w // CHUNK
    assert per_w % CHUNK == 0 and CHUNK % 8 == 0 and n_chunks % NBUF == 0

    mesh = plsc.VectorSubcoreMesh(core_axis_name="c", subcore_axis_name="s")

    @functools.partial(
        pl.kernel,
        out_type=jax.ShapeDtypeStruct((N, D), jnp.float32),
        mesh=mesh,
        scratch_types=[
            [pltpu.VMEM((CHUNK,), jnp.int32) for _ in range(NBUF)],
            [pltpu.VMEM((CHUNK, D), jnp.float32) for _ in range(NBUF)],
            [pltpu.SemaphoreType.DMA for _ in range(NBUF)],
            [pltpu.SemaphoreType.DMA for _ in range(NBUF)],
            [pltpu.SemaphoreType.DMA for _ in range(NBUF)],
        ],
        compiler_params=pltpu.CompilerParams(use_tc_tiling_on_sc=False),
    )
    def gather_kernel(x_hbm, table_hbm, out_hbm, idxs, rows, sis, sgs, sos):
        wid = lax.axis_index("s") * NC + lax.axis_index("c")
        base = wid * per_w

        def start_idx(chunk, b):
            off = base + chunk * CHUNK
            pltpu.async_copy(x_hbm.at[pl.ds(off, CHUNK)], idxs[b], sis[b])

        for b in range(NBUF):
            start_idx(b, b)

        def body(g, carry):
            for b in range(NBUF):
                i = NBUF * g + b
                off = base + i * CHUNK
                pltpu.make_async_copy(
                    x_hbm.at[pl.ds(base, CHUNK)], idxs[b], sis[b]).wait()

                @pl.when(g > 0)
                def _():
                    pltpu.make_async_copy(
                        rows[b], out_hbm.at[pl.ds(base, CHUNK)], sos[b]).wait()

                pltpu.async_copy(table_hbm.at[idxs[b]], rows[b], sgs[b]).wait()

                @pl.when(i + NBUF < n_chunks)
                def _():
                    start_idx(i + NBUF, b)

                pltpu.async_copy(rows[b], out_hbm.at[pl.ds(off, CHUNK)], sos[b])
            return carry

        lax.fori_loop(0, n_chunks // NBUF, body, 0)

        for b in range(NBUF):
            pltpu.make_async_copy(
                rows[b], out_hbm.at[pl.ds(base, CHUNK)], sos[b]).wait()

    return gather_kernel


def kernel(x, table):
    B, S = x.shape
    V, D = table.shape
    N = B * S
    info = plsc.get_sparse_core_info()
    f = _build_gather(N, V, D, info.num_cores, info.num_subcores)
    out = f(x.reshape(N), table)
    return out.reshape(B, S, D)


# speedup vs baseline: 1.4939x; 1.0010x over previous
import functools

import jax
import jax.numpy as jnp
from jax import lax
from jax.experimental import pallas as pl
from jax.experimental.pallas import tpu as pltpu
from jax.experimental.pallas import tpu_sc as plsc

NBUF = 2
NSTREAM = 8


@functools.cache
def _build_gather(N, V, D, NC, NS):
    NW = NC * NS
    per_w = N // NW
    CHUNK = 1600
    SUB = CHUNK // NSTREAM
    n_chunks = per_w // CHUNK
    assert per_w % CHUNK == 0 and CHUNK % 8 == 0 and n_chunks % NBUF == 0
    assert CHUNK % NSTREAM == 0 and SUB % 8 == 0

    mesh = plsc.VectorSubcoreMesh(core_axis_name="c", subcore_axis_name="s")

    @functools.partial(
        pl.kernel,
        out_type=jax.ShapeDtypeStruct((N, D), jnp.float32),
        mesh=mesh,
        scratch_types=[
            [pltpu.VMEM((CHUNK,), jnp.int32) for _ in range(NBUF)],
            [pltpu.VMEM((CHUNK, D), jnp.float32) for _ in range(NBUF)],
            [pltpu.SemaphoreType.DMA for _ in range(NBUF)],
            [pltpu.SemaphoreType.DMA for _ in range(NBUF)],
            [pltpu.SemaphoreType.DMA for _ in range(NBUF)],
        ],
        compiler_params=pltpu.CompilerParams(use_tc_tiling_on_sc=False),
    )
    def gather_kernel(x_hbm, table_hbm, out_hbm, idxs, rows, sis, sgs, sos):
        wid = lax.axis_index("s") * NC + lax.axis_index("c")
        base = wid * per_w

        def start_idx(chunk, b):
            off = base + chunk * CHUNK
            pltpu.async_copy(x_hbm.at[pl.ds(off, CHUNK)], idxs[b], sis[b])

        for b in range(NBUF):
            start_idx(b, b)

        def body(g, carry):
            for b in range(NBUF):
                i = NBUF * g + b
                off = base + i * CHUNK
                pltpu.make_async_copy(
                    x_hbm.at[pl.ds(base, CHUNK)], idxs[b], sis[b]).wait()

                @pl.when(g > 0)
                def _():
                    pltpu.make_async_copy(
                        rows[b], out_hbm.at[pl.ds(base, CHUNK)], sos[b]).wait()

                descs = [
                    pltpu.async_copy(
                        table_hbm.at[idxs[b].at[pl.ds(k * SUB, SUB)]],
                        rows[b].at[pl.ds(k * SUB, SUB)],
                        sgs[b])
                    for k in range(NSTREAM)
                ]
                for d in descs:
                    d.wait()

                @pl.when(i + NBUF < n_chunks)
                def _():
                    start_idx(i + NBUF, b)

                pltpu.async_copy(rows[b], out_hbm.at[pl.ds(off, CHUNK)], sos[b])
            return carry

        lax.fori_loop(0, n_chunks // NBUF, body, 0)

        for b in range(NBUF):
            pltpu.make_async_copy(
                rows[b], out_hbm.at[pl.ds(base, CHUNK)], sos[b]).wait()

    return gather_kernel


def kernel(x, table):
    B, S = x.shape
    V, D = table.shape
    N = B * S
    info = plsc.get_sparse_core_info()
    f = _build_gather(N, V, D, info.num_cores, info.num_subcores)
    out = f(x.reshape(N), table)
    return out.reshape(B, S, D)


# speedup vs baseline: 1.5688x; 1.0502x over previous
import functools

import jax
import jax.numpy as jnp
from jax import lax
from jax.experimental import pallas as pl
from jax.experimental.pallas import tpu as pltpu
from jax.experimental.pallas import tpu_sc as plsc

NBUF = 2
NSTREAM = 8


@functools.cache
def _build_gather(N, V, D, NC, NS):
    NW = NC * NS
    per_w = N // NW
    CHUNK = 1600
    SUB = CHUNK // NSTREAM
    n_chunks = per_w // CHUNK
    assert per_w % CHUNK == 0 and CHUNK % 8 == 0 and n_chunks % NBUF == 0
    assert CHUNK % NSTREAM == 0 and SUB % 8 == 0

    mesh = plsc.VectorSubcoreMesh(core_axis_name="c", subcore_axis_name="s")

    @functools.partial(
        pl.kernel,
        out_type=jax.ShapeDtypeStruct((N, D), jnp.float32),
        mesh=mesh,
        scratch_types=[
            [pltpu.VMEM((CHUNK,), jnp.int32) for _ in range(NBUF)],
            [pltpu.VMEM((CHUNK, D), jnp.float32) for _ in range(NBUF)],
            [pltpu.SemaphoreType.DMA for _ in range(NBUF)],
            [pltpu.SemaphoreType.DMA for _ in range(NBUF)],
            [pltpu.SemaphoreType.DMA for _ in range(NBUF)],
        ],
        compiler_params=pltpu.CompilerParams(use_tc_tiling_on_sc=False),
    )
    def gather_kernel(x_hbm, table_hbm, out_hbm, idxs, rows, sis, sgs, sos):
        wid = lax.axis_index("s") * NC + lax.axis_index("c")
        base = wid * per_w

        def start_idx(chunk, b):
            off = base + chunk * CHUNK
            pltpu.async_copy(x_hbm.at[pl.ds(off, CHUNK)], idxs[b], sis[b])

        for b in range(NBUF):
            start_idx(b, b)

        def body(g, carry):
            for b in range(NBUF):
                i = NBUF * g + b
                off = base + i * CHUNK
                pltpu.make_async_copy(
                    x_hbm.at[pl.ds(base, CHUNK)], idxs[b], sis[b]).wait()

                @pl.when(g > 0)
                def _():
                    pltpu.make_async_copy(
                        rows[b], out_hbm.at[pl.ds(base, CHUNK)], sos[b]).wait()

                descs = [
                    pltpu.async_copy(
                        table_hbm.at[idxs[b].at[pl.ds(k * SUB, SUB)]],
                        rows[b].at[pl.ds(k * SUB, SUB)],
                        sgs[b])
                    for k in range(NSTREAM)
                ]
                for d in descs:
                    d.wait()

                @pl.when(i + NBUF < n_chunks)
                def _():
                    start_idx(i + NBUF, b)

                pltpu.async_copy(rows[b], out_hbm.at[pl.ds(off, CHUNK)], sos[b])
            return carry

        lax.fori_loop(0, n_chunks // NBUF, body, 0)

        for b in range(NBUF):
            pltpu.make_async_copy(
                rows[b], out_hbm.at[pl.ds(base, CHUNK)], sos[b]).wait()

    return gather_kernel


def kernel(x, table):
    B, S = x.shape
    V, D = table.shape
    N = B * S
    info = plsc.get_sparse_core_info()
    f = _build_gather(N, V, D, info.num_cores, info.num_subcores)
    xt_flat = jnp.transpose(x).reshape(N)
    out_flat = f(xt_flat, table)
    return out_flat.reshape(S, B, D).transpose(1, 0, 2)
